# baseline (speedup 1.0000x reference)
import functools

import jax
import jax.numpy as jnp
from jax import lax
from jax.experimental import pallas as pl
from jax.experimental.pallas import tpu as pltpu
from jax.experimental.pallas import tpu_sc as plsc

B = 4096
L = 200
EMBED = 32
POS = 200
LANES = 16

NUM_CORES = 2
NUM_SUBCORES = 16
NW = NUM_CORES * NUM_SUBCORES
BPW = B // NW
HALF = BPW // 2
C0 = 104
C1 = L - C0


def _sc_pool(x1, x2, x3, x4, e1, e2):
  mesh = plsc.VectorSubcoreMesh(core_axis_name="c", subcore_axis_name="s")

  @functools.partial(
      pl.kernel,
      mesh=mesh,
      compiler_params=pltpu.CompilerParams(
          use_tc_tiling_on_sc=False, needs_layout_passes=False),
      out_type=(
          jax.ShapeDtypeStruct((B, EMBED), jnp.float32),
          jax.ShapeDtypeStruct((B, EMBED), jnp.float32),
          jax.ShapeDtypeStruct((B,), jnp.float32),
      ),
      scratch_types=[
          pltpu.VMEM((HALF * L,), jnp.int32),
          pltpu.VMEM((HALF * L,), jnp.int32),
          pltpu.VMEM((HALF, L), jnp.int32),
          pltpu.VMEM((HALF, L), jnp.float32),
          pltpu.VMEM((L, EMBED), jnp.float32),
          pltpu.VMEM((L, EMBED), jnp.float32),
          pltpu.VMEM((L, EMBED), jnp.float32),
          pltpu.VMEM((L, EMBED), jnp.float32),
          pltpu.VMEM((POS + 1, 1), jnp.float32),
          pltpu.VMEM((HALF, EMBED), jnp.float32),
          pltpu.VMEM((HALF, EMBED), jnp.float32),
          pltpu.VMEM((HALF,), jnp.float32),
          pltpu.SemaphoreType.DMA,
          pltpu.SemaphoreType.DMA,
      ],
  )
  def pool(x1h, x2h, x3h, x4h, e1h, e2h, s1h, s2h, s3h,
           x1s, x2s, x3s, x4s, r1a, r1b, r2a, r2b, e2v, s1a, s2a, s3a,
           sem0, sem1):
    wid = lax.axis_index("s") * NUM_CORES + lax.axis_index("c")
    pltpu.sync_copy(e2h, e2v)
    lane = lax.iota(jnp.int32, LANES)
    zeros_i = jnp.zeros((LANES,), jnp.int32)
    for ph in range(2):
      base = wid * BPW + ph * HALF
      pltpu.sync_copy(x1h.at[pl.ds(base * L, HALF * L)], x1s)
      pltpu.sync_copy(x2h.at[pl.ds(base * L, HALF * L)], x2s)
      pltpu.sync_copy(x3h.at[pl.ds(base, HALF)], x3s)
      pltpu.sync_copy(x4h.at[pl.ds(base, HALF)], x4s)

      def remap_body(k, carry):
        off = pl.multiple_of(k * LANES, LANES)
        for xs in (x1s, x2s):
          v = xs[pl.ds(off, LANES)]
          blk = ((v >= 250000).astype(jnp.int32)
                 + (v >= 500000).astype(jnp.int32)
                 + (v >= 750000).astype(jnp.int32))
          xs[pl.ds(off, LANES)] = (v - blk * 250000) * 4 + blk
        return carry

      lax.fori_loop(0, (HALF * L) // LANES, remap_body, 0, unroll=4)

      for g0 in range(0, HALF, LANES):
        bvec = g0 + lane

        def t3_body(j, acc):
          jvec = jnp.full((LANES,), j, jnp.int32)
          pos = plsc.load_gather(x3s, [bvec, jvec])
          vals = plsc.load_gather(e2v, [pos, zeros_i])
          return acc + vals

        acc3 = lax.fori_loop(0, L, t3_body, jnp.zeros((LANES,), jnp.float32),
                             unroll=8)
        s3a[pl.ds(g0, LANES)] = acc3

      def chunk_copies(gi, r1, r2, sem):
        off = pl.multiple_of(gi * L, 8)
        out = []
        for (idxs, rows) in ((x1s, r1), (x2s, r2)):
          out.append(pltpu.make_async_copy(
              e1h.at[idxs.at[pl.ds(off, C0)]], rows.at[pl.ds(0, C0)], sem))
          out.append(pltpu.make_async_copy(
              e1h.at[idxs.at[pl.ds(off + C0, C1)]], rows.at[pl.ds(C0, C1)],
              sem))
        return out

      def fire(gi, r1, r2, sem):
        for cp in chunk_copies(gi, r1, r2, sem):
          cp.start()

      def drain(gi, r1, r2, sem):
        for cp in chunk_copies(gi, r1, r2, sem):
          cp.wait()

      zf = jnp.zeros((LANES,), jnp.float32)

      def accum_rows(jbase, jlo, jhi, w16, r1, r2, accs):
        a10, a11, a20, a21 = accs
        for jj in range(jlo, jhi):
          j = jbase + jj
          w = jnp.take_along_axis(
              w16, jnp.full((LANES,), jj, jnp.int32), axis=0)
          a10 = a10 + r1[j, pl.ds(0, LANES)] * w
          a11 = a11 + r1[j, pl.ds(LANES, LANES)] * w
          a20 = a20 + r2[j, pl.ds(0, LANES)]
          a21 = a21 + r2[j, pl.ds(LANES, LANES)]
        return (a10, a11, a20, a21)

      def accumulate(gi, r1, r2):
        def chunk_body(k, accs):
          off = pl.multiple_of(k * LANES, LANES)
          w16 = x4s[gi, pl.ds(off, LANES)]
          return accum_rows(k * LANES, 0, LANES, w16, r1, r2, accs)

        accs = lax.fori_loop(0, L // LANES, chunk_body, (zf, zf, zf, zf))
        w16 = x4s[gi, pl.ds(L - LANES, LANES)]
        a10, a11, a20, a21 = accum_rows(L - LANES, 8, LANES, w16, r1, r2, accs)
        s1a[gi, pl.ds(0, LANES)] = a10
        s1a[gi, pl.ds(LANES, LANES)] = a11
        s2a[gi, pl.ds(0, LANES)] = a20
        s2a[gi, pl.ds(LANES, LANES)] = a21

      fire(0, r1a, r2a, sem0)

      def pair_body(i, carry):
        gi0 = i * 2
        drain(gi0, r1a, r2a, sem0)
        fire(gi0 + 1, r1b, r2b, sem1)
        accumulate(gi0, r1a, r2a)

        @pl.when(gi0 + 2 < HALF)
        def _():
          fire(gi0 + 2, r1a, r2a, sem0)

        drain(gi0 + 1, r1b, r2b, sem1)
        accumulate(gi0 + 1, r1b, r2b)
        return carry

      lax.fori_loop(0, HALF // 2, pair_body, 0)

      pltpu.sync_copy(s1a, s1h.at[pl.ds(base, HALF)])
      pltpu.sync_copy(s2a, s2h.at[pl.ds(base, HALF)])
      pltpu.sync_copy(s3a, s3h.at[pl.ds(base, HALF)])

  return pool(x1, x2, x3, x4, e1, e2)


def _tc_tail(s1, s2, s3, t1b, w1, bb1, t2b, w2, bb2):
  def body(s1r, s2r, s3r, t1br, w1r, b1r, t2br, w2r, b2r, outr):
    h1 = jnp.tanh(s1r[...] + t1br[...][None, :])
    h1 = jnp.tanh(
        lax.dot_general(h1, w1r[...], (((1,), (1,)), ((), ())),
                        preferred_element_type=jnp.float32) + b1r[...][None, :])
    h2 = jnp.tanh(s2r[...] + t2br[...][None, :])
    h2 = jnp.tanh(
        lax.dot_general(h2, w2r[...], (((1,), (1,)), ((), ())),
                        preferred_element_type=jnp.float32) + b2r[...][None, :])
    x12 = jax.nn.sigmoid(jnp.sum(h1 * h2, axis=1))
    outr[...] = x12 * jax.nn.sigmoid(s3r[...])

  return pl.pallas_call(
      body,
      out_shape=jax.ShapeDtypeStruct((B,), jnp.float32),
  )(s1, s2, s3, t1b, w1, bb1, t2b, w2, bb2)


def kernel(x1, x2, x3, x4, E1, t1_bias1, W1, b1, t2_bias1, W2, b2, E2):
  q = E1.shape[0] // 4
  e1k = lax.optimization_barrier(
      E1.reshape(4, q, EMBED).transpose(1, 0, 2).reshape(q, 4 * EMBED)
  ).reshape(E1.shape)
  s1, s2, s3 = _sc_pool(x1.astype(jnp.int32).reshape(-1),
                        x2.astype(jnp.int32).reshape(-1),
                        x3.astype(jnp.int32), x4, e1k, E2)
  return _tc_tail(s1, s2, s3, t1_bias1, W1, b1, t2_bias1, W2, b2)


# speedup vs baseline: 1.8052x; 1.8052x over previous
import functools

import jax
import jax.numpy as jnp
from jax import lax
from jax.experimental import pallas as pl
from jax.experimental.pallas import tpu as pltpu
from jax.experimental.pallas import tpu_sc as plsc

B = 4096
L = 200
EMBED = 32
POS = 200
LANES = 16

NUM_CORES = 2
NUM_SUBCORES = 16
NW = NUM_CORES * NUM_SUBCORES
BPW = B // NW
HALF = BPW // 2
C0 = 104
C1 = L - C0


def _sc_pool(x1, x2, x3, x4, e1, e2):
  mesh = plsc.VectorSubcoreMesh(core_axis_name="c", subcore_axis_name="s")

  @functools.partial(
      pl.kernel,
      mesh=mesh,
      compiler_params=pltpu.CompilerParams(
          use_tc_tiling_on_sc=False, needs_layout_passes=False),
      out_type=(
          jax.ShapeDtypeStruct((B, EMBED), jnp.float32),
          jax.ShapeDtypeStruct((B, EMBED), jnp.float32),
          jax.ShapeDtypeStruct((B,), jnp.float32),
      ),
      scratch_types=[
          pltpu.VMEM((HALF * L,), jnp.int32),
          pltpu.VMEM((HALF * L,), jnp.int32),
          pltpu.VMEM((HALF, L), jnp.int32),
          pltpu.VMEM((HALF, L), jnp.float32),
          pltpu.VMEM((L, EMBED), jnp.float32),
          pltpu.VMEM((L, EMBED), jnp.float32),
          pltpu.VMEM((L, EMBED), jnp.float32),
          pltpu.VMEM((L, EMBED), jnp.float32),
          pltpu.VMEM((POS + 1, 1), jnp.float32),
          pltpu.VMEM((HALF, EMBED), jnp.float32),
          pltpu.VMEM((HALF, EMBED), jnp.float32),
          pltpu.VMEM((HALF,), jnp.float32),
          pltpu.SemaphoreType.DMA,
          pltpu.SemaphoreType.DMA,
      ],
  )
  def pool(x1h, x2h, x3h, x4h, e1h, e2h, s1h, s2h, s3h,
           x1s, x2s, x3s, x4s, r1a, r1b, r2a, r2b, e2v, s1a, s2a, s3a,
           sem0, sem1):
    wid = lax.axis_index("s") * NUM_CORES + lax.axis_index("c")
    pltpu.sync_copy(e2h, e2v)
    lane = lax.iota(jnp.int32, LANES)
    zeros_i = jnp.zeros((LANES,), jnp.int32)
    for ph in range(2):
      base = wid * BPW + ph * HALF
      pltpu.sync_copy(x1h.at[pl.ds(base * L, HALF * L)], x1s)
      pltpu.sync_copy(x2h.at[pl.ds(base * L, HALF * L)], x2s)
      pltpu.sync_copy(x3h.at[pl.ds(base, HALF)], x3s)
      pltpu.sync_copy(x4h.at[pl.ds(base, HALF)], x4s)

      def remap_body(k, carry):
        off = pl.multiple_of(k * LANES, LANES)
        for xs in (x1s, x2s):
          v = xs[pl.ds(off, LANES)]
          blk = ((v >= 250000).astype(jnp.int32)
                 + (v >= 500000).astype(jnp.int32)
                 + (v >= 750000).astype(jnp.int32))
          xs[pl.ds(off, LANES)] = (v - blk * 250000) * 4 + blk
        return carry

      lax.fori_loop(0, (HALF * L) // LANES, remap_body, 0, unroll=4)

      for g0 in range(0, HALF, LANES):
        bvec = g0 + lane

        def t3_body(j, acc):
          jvec = jnp.full((LANES,), j, jnp.int32)
          pos = plsc.load_gather(x3s, [bvec, jvec])
          vals = plsc.load_gather(e2v, [pos, zeros_i])
          return acc + vals

        acc3 = lax.fori_loop(0, L, t3_body, jnp.zeros((LANES,), jnp.float32),
                             unroll=8)
        s3a[pl.ds(g0, LANES)] = acc3

      def chunk_copies(gi, r1, r2, sem):
        off = pl.multiple_of(gi * L, 8)
        out = []
        for (idxs, rows) in ((x1s, r1), (x2s, r2)):
          out.append(pltpu.make_async_copy(
              e1h.at[idxs.at[pl.ds(off, C0)]], rows.at[pl.ds(0, C0)], sem))
          out.append(pltpu.make_async_copy(
              e1h.at[idxs.at[pl.ds(off + C0, C1)]], rows.at[pl.ds(C0, C1)],
              sem))
        return out

      def fire(gi, r1, r2, sem):
        for cp in chunk_copies(gi, r1, r2, sem):
          cp.start()

      def drain(gi, r1, r2, sem):
        for cp in chunk_copies(gi, r1, r2, sem):
          cp.wait()

      zf = jnp.zeros((LANES,), jnp.float32)

      def accum_rows(jbase, jlo, jhi, w16, r1, r2, accs):
        a10, a11, a20, a21 = accs
        for jj in range(jlo, jhi):
          j = jbase + jj
          w = jnp.take_along_axis(
              w16, jnp.full((LANES,), jj, jnp.int32), axis=0)
          a10 = a10 + r1[j, pl.ds(0, LANES)] * w
          a11 = a11 + r1[j, pl.ds(LANES, LANES)] * w
          a20 = a20 + r2[j, pl.ds(0, LANES)]
          a21 = a21 + r2[j, pl.ds(LANES, LANES)]
        return (a10, a11, a20, a21)

      def accumulate(gi, r1, r2):
        def chunk_body(k, accs):
          off = pl.multiple_of(k * LANES, LANES)
          w16 = x4s[gi, pl.ds(off, LANES)]
          return accum_rows(k * LANES, 0, LANES, w16, r1, r2, accs)

        accs = lax.fori_loop(0, L // LANES, chunk_body, (zf, zf, zf, zf))
        w16 = x4s[gi, pl.ds(L - LANES, LANES)]
        a10, a11, a20, a21 = accum_rows(L - LANES, 8, LANES, w16, r1, r2, accs)
        s1a[gi, pl.ds(0, LANES)] = a10
        s1a[gi, pl.ds(LANES, LANES)] = a11
        s2a[gi, pl.ds(0, LANES)] = a20
        s2a[gi, pl.ds(LANES, LANES)] = a21

      fire(0, r1a, r2a, sem0)

      def pair_body(i, carry):
        gi0 = i * 2
        drain(gi0, r1a, r2a, sem0)
        fire(gi0 + 1, r1b, r2b, sem1)
        accumulate(gi0, r1a, r2a)

        @pl.when(gi0 + 2 < HALF)
        def _():
          fire(gi0 + 2, r1a, r2a, sem0)

        drain(gi0 + 1, r1b, r2b, sem1)
        accumulate(gi0 + 1, r1b, r2b)
        return carry

      lax.fori_loop(0, HALF // 2, pair_body, 0)

      pltpu.sync_copy(s1a, s1h.at[pl.ds(base, HALF)])
      pltpu.sync_copy(s2a, s2h.at[pl.ds(base, HALF)])
      pltpu.sync_copy(s3a, s3h.at[pl.ds(base, HALF)])

  return pool(x1, x2, x3, x4, e1, e2)


def _tc_tail(s1, s2, s3, t1b, w1, bb1, t2b, w2, bb2):
  def body(s1r, s2r, s3r, t1br, w1r, b1r, t2br, w2r, b2r, outr):
    h1 = jnp.tanh(s1r[...] + t1br[...][None, :])
    h1 = jnp.tanh(
        lax.dot_general(h1, w1r[...], (((1,), (1,)), ((), ())),
                        preferred_element_type=jnp.float32) + b1r[...][None, :])
    h2 = jnp.tanh(s2r[...] + t2br[...][None, :])
    h2 = jnp.tanh(
        lax.dot_general(h2, w2r[...], (((1,), (1,)), ((), ())),
                        preferred_element_type=jnp.float32) + b2r[...][None, :])
    x12 = jax.nn.sigmoid(jnp.sum(h1 * h2, axis=1))
    outr[...] = x12 * jax.nn.sigmoid(s3r[...])

  return pl.pallas_call(
      body,
      out_shape=jax.ShapeDtypeStruct((B,), jnp.float32),
  )(s1, s2, s3, t1b, w1, bb1, t2b, w2, bb2)


def kernel(x1, x2, x3, x4, E1, t1_bias1, W1, b1, t2_bias1, W2, b2, E2):
  q = E1.shape[0] // 4
  f = jnp.empty((q, 4 * EMBED), jnp.float32)
  for k in range(4):
    f = lax.dynamic_update_slice(f, E1[k * q:(k + 1) * q, :], (0, k * EMBED))
  e1k = lax.optimization_barrier(f).reshape(E1.shape)
  s1, s2, s3 = _sc_pool(x1.astype(jnp.int32).reshape(-1),
                        x2.astype(jnp.int32).reshape(-1),
                        x3.astype(jnp.int32), x4, e1k, E2)
  return _tc_tail(s1, s2, s3, t1_bias1, W1, b1, t2_bias1, W2, b2)


# speedup vs baseline: 1.8071x; 1.0011x over previous
import functools

import jax
import jax.numpy as jnp
from jax import lax
from jax.experimental import pallas as pl
from jax.experimental.pallas import tpu as pltpu
from jax.experimental.pallas import tpu_sc as plsc

B = 4096
L = 200
EMBED = 32
POS = 200
LANES = 16

NUM_CORES = 2
NUM_SUBCORES = 16
NW = NUM_CORES * NUM_SUBCORES
BPW = B // NW
HALF = BPW // 2
C0 = 104
C1 = L - C0


def _sc_pool(x1, x2, x3, x4, e1, e2):
  mesh = plsc.VectorSubcoreMesh(core_axis_name="c", subcore_axis_name="s")

  @functools.partial(
      pl.kernel,
      mesh=mesh,
      compiler_params=pltpu.CompilerParams(
          use_tc_tiling_on_sc=False, needs_layout_passes=False),
      out_type=(
          jax.ShapeDtypeStruct((B, EMBED), jnp.float32),
          jax.ShapeDtypeStruct((B, EMBED), jnp.float32),
          jax.ShapeDtypeStruct((B,), jnp.float32),
      ),
      scratch_types=[
          pltpu.VMEM((HALF * L,), jnp.int32),
          pltpu.VMEM((HALF * L,), jnp.int32),
          pltpu.VMEM((HALF, L), jnp.int32),
          pltpu.VMEM((HALF, L), jnp.float32),
          pltpu.VMEM((L, EMBED), jnp.float32),
          pltpu.VMEM((L, EMBED), jnp.float32),
          pltpu.VMEM((L, EMBED), jnp.float32),
          pltpu.VMEM((L, EMBED), jnp.float32),
          pltpu.VMEM((POS + 1, 1), jnp.float32),
          pltpu.VMEM((HALF, EMBED), jnp.float32),
          pltpu.VMEM((HALF, EMBED), jnp.float32),
          pltpu.VMEM((HALF,), jnp.float32),
          pltpu.SemaphoreType.DMA,
          pltpu.SemaphoreType.DMA,
      ],
  )
  def pool(x1h, x2h, x3h, x4h, e1h, e2h, s1h, s2h, s3h,
           x1s, x2s, x3s, x4s, r1a, r1b, r2a, r2b, e2v, s1a, s2a, s3a,
           sem0, sem1):
    wid = lax.axis_index("s") * NUM_CORES + lax.axis_index("c")
    pltpu.sync_copy(e2h, e2v)
    lane = lax.iota(jnp.int32, LANES)
    zeros_i = jnp.zeros((LANES,), jnp.int32)
    for ph in range(2):
      base = wid * BPW + ph * HALF
      pltpu.sync_copy(x1h.at[pl.ds(base * L, HALF * L)], x1s)
      pltpu.sync_copy(x2h.at[pl.ds(base * L, HALF * L)], x2s)
      pltpu.sync_copy(x3h.at[pl.ds(base, HALF)], x3s)
      pltpu.sync_copy(x4h.at[pl.ds(base, HALF)], x4s)

      def remap_body(k, carry):
        off = pl.multiple_of(k * LANES, LANES)
        for xs in (x1s, x2s):
          v = xs[pl.ds(off, LANES)]
          blk = ((v >= 250000).astype(jnp.int32)
                 + (v >= 500000).astype(jnp.int32)
                 + (v >= 750000).astype(jnp.int32))
          xs[pl.ds(off, LANES)] = (v - blk * 250000) * 4 + blk
        return carry

      lax.fori_loop(0, (HALF * L) // LANES, remap_body, 0, unroll=4)

      for g0 in range(0, HALF, LANES):
        bvec = g0 + lane

        def t3_body(j, acc):
          jvec = jnp.full((LANES,), j, jnp.int32)
          pos = plsc.load_gather(x3s, [bvec, jvec])
          vals = plsc.load_gather(e2v, [pos, zeros_i])
          return acc + vals

        acc3 = lax.fori_loop(0, L, t3_body, jnp.zeros((LANES,), jnp.float32),
                             unroll=8)
        s3a[pl.ds(g0, LANES)] = acc3

      def chunk_copies(gi, r1, r2, sem):
        off = pl.multiple_of(gi * L, 8)
        out = []
        for (idxs, rows) in ((x1s, r1), (x2s, r2)):
          out.append(pltpu.make_async_copy(
              e1h.at[idxs.at[pl.ds(off, C0)]], rows.at[pl.ds(0, C0)], sem))
          out.append(pltpu.make_async_copy(
              e1h.at[idxs.at[pl.ds(off + C0, C1)]], rows.at[pl.ds(C0, C1)],
              sem))
        return out

      def fire(gi, r1, r2, sem):
        for cp in chunk_copies(gi, r1, r2, sem):
          cp.start()

      def drain(gi, r1, r2, sem):
        for cp in chunk_copies(gi, r1, r2, sem):
          cp.wait()

      zf = jnp.zeros((LANES,), jnp.float32)

      def accum_rows(jbase, jlo, jhi, w16, r1, r2, accs):
        a10, a11, a20, a21 = accs
        for jj in range(jlo, jhi):
          j = jbase + jj
          w = jnp.take_along_axis(
              w16, jnp.full((LANES,), jj, jnp.int32), axis=0)
          a10 = a10 + r1[j, pl.ds(0, LANES)] * w
          a11 = a11 + r1[j, pl.ds(LANES, LANES)] * w
          a20 = a20 + r2[j, pl.ds(0, LANES)]
          a21 = a21 + r2[j, pl.ds(LANES, LANES)]
        return (a10, a11, a20, a21)

      def accumulate(gi, r1, r2):
        def chunk_body(k, accs):
          off = pl.multiple_of(k * LANES, LANES)
          w16 = x4s[gi, pl.ds(off, LANES)]
          return accum_rows(k * LANES, 0, LANES, w16, r1, r2, accs)

        accs = lax.fori_loop(0, L // LANES, chunk_body, (zf, zf, zf, zf))
        w16 = x4s[gi, pl.ds(L - LANES, LANES)]
        a10, a11, a20, a21 = accum_rows(L - LANES, 8, LANES, w16, r1, r2, accs)
        s1a[gi, pl.ds(0, LANES)] = a10
        s1a[gi, pl.ds(LANES, LANES)] = a11
        s2a[gi, pl.ds(0, LANES)] = a20
        s2a[gi, pl.ds(LANES, LANES)] = a21

      fire(0, r1a, r2a, sem0)

      def pair_body(i, carry):
        gi0 = i * 2
        drain(gi0, r1a, r2a, sem0)
        fire(gi0 + 1, r1b, r2b, sem1)
        accumulate(gi0, r1a, r2a)

        @pl.when(gi0 + 2 < HALF)
        def _():
          fire(gi0 + 2, r1a, r2a, sem0)

        drain(gi0 + 1, r1b, r2b, sem1)
        accumulate(gi0 + 1, r1b, r2b)
        return carry

      lax.fori_loop(0, HALF // 2, pair_body, 0)

      pltpu.sync_copy(s1a, s1h.at[pl.ds(base, HALF)])
      pltpu.sync_copy(s2a, s2h.at[pl.ds(base, HALF)])
      pltpu.sync_copy(s3a, s3h.at[pl.ds(base, HALF)])

  return pool(x1, x2, x3, x4, e1, e2)


def _tc_tail(s1, s2, s3, t1b, w1, bb1, t2b, w2, bb2):
  def body(s1r, s2r, s3r, t1br, w1r, b1r, t2br, w2r, b2r, outr):
    h1 = jnp.tanh(s1r[...] + t1br[...][None, :])
    h1 = jnp.tanh(
        lax.dot_general(h1, w1r[...], (((1,), (1,)), ((), ())),
                        preferred_element_type=jnp.float32) + b1r[...][None, :])
    h2 = jnp.tanh(s2r[...] + t2br[...][None, :])
    h2 = jnp.tanh(
        lax.dot_general(h2, w2r[...], (((1,), (1,)), ((), ())),
                        preferred_element_type=jnp.float32) + b2r[...][None, :])
    x12 = jax.nn.sigmoid(jnp.sum(h1 * h2, axis=1))
    outr[...] = x12 * jax.nn.sigmoid(s3r[...])

  return pl.pallas_call(
      body,
      out_shape=jax.ShapeDtypeStruct((B,), jnp.float32),
  )(s1, s2, s3, t1b, w1, bb1, t2b, w2, bb2)


def kernel(x1, x2, x3, x4, E1, t1_bias1, W1, b1, t2_bias1, W2, b2, E2):
  q = E1.shape[0] // 4
  f = jnp.empty((q, 4 * EMBED), jnp.float32)
  for k in range(4):
    f = lax.dynamic_update_slice(f, E1[k * q:(k + 1) * q, :], (0, k * EMBED))
  e1k = f.reshape(E1.shape)
  s1, s2, s3 = _sc_pool(x1.astype(jnp.int32).reshape(-1),
                        x2.astype(jnp.int32).reshape(-1),
                        x3.astype(jnp.int32), x4, e1k, E2)
  return _tc_tail(s1, s2, s3, t1_bias1, W1, b1, t2_bias1, W2, b2)
